# baseline (speedup 1.0000x reference)
import functools
import math

import jax
import jax.numpy as jnp
from jax import lax
from jax.experimental import pallas as pl
from jax.experimental.pallas import tpu as pltpu
from jax.experimental.pallas import tpu_sc as plsc

_BN_EPS = 1e-5
_K = 9
_HIGH = lax.Precision.HIGHEST


_ROW_BLOCK = 256


def _fc1_kernel(x_ref, w1_ref, b1_ref, g1_ref, be1_ref, at_ref, wbt_ref,
                xn_ref, ut_ref, vt_ref):
    xb = x_ref[0]
    h0 = lax.dot_general(w1_ref[...].astype(jnp.bfloat16),
                         xb.astype(jnp.bfloat16), (((1,), (0,)), ((), ())),
                         preferred_element_type=jnp.float32) + b1_ref[...]
    h = h0 / jnp.sqrt(jnp.float32(1.0 + _BN_EPS)) * g1_ref[...] + be1_ref[...]
    norm = jnp.sqrt(jnp.sum(h * h, axis=0, keepdims=True))
    xn_ref[0] = h / jnp.maximum(norm, 1e-12)
    dn = (((0,), (0,)), ((), ()))
    ut_ref[0] = lax.dot_general(h, at_ref[...], dn, precision=_HIGH)
    vt_ref[0] = lax.dot_general(h, wbt_ref[...], dn, precision=_HIGH)


def _knn_kernel(xnr_ref, xnf_ref, idx_ref, *, k, n_real, n_tot):
    xr = xnr_ref[0]
    xf = xnf_ref[0]
    C, R = xr.shape
    N = xf.shape[1]
    dn = (((0,), (0,)), ((), ()))
    s = lax.dot_general(xr.astype(jnp.bfloat16), xf.astype(jnp.bfloat16), dn,
                        preferred_element_type=jnp.float32)
    xsq_col = jnp.sum(xf * xf, axis=0, keepdims=True)
    ones = jnp.ones((C, 1), jnp.float32)
    xsq_row = lax.dot_general(xr * xr, ones, dn, precision=_HIGH)
    dist = (xsq_row + (-2.0) * s) + xsq_col
    iota = lax.broadcasted_iota(jnp.int32, (R, N), 1)
    if n_real < N:
        dist = jnp.where(iota < n_real, dist, jnp.float32(jnp.inf))
    big = jnp.int32(2**30)
    inf = jnp.float32(jnp.inf)
    cols = []
    for _ in range(k):
        mn = jnp.min(dist, axis=1, keepdims=True)
        idx = jnp.min(jnp.where(dist <= mn, iota, big), axis=1, keepdims=True)
        cols.append(idx)
        dist = jnp.where(iota == idx, inf, dist)
    base = pl.program_id(0) * n_tot
    idx_ref[0] = jnp.concatenate(cols, axis=1) + base


def _gather_max_sc(table, idx, k):
    T, D = table.shape
    M = idx.shape[0] // k
    info = plsc.get_sparse_core_info()
    nw = info.num_cores * info.num_subcores
    npw = M // nw
    ch = 8
    nch = npw // ch
    chi = ch * k
    mesh = plsc.VectorSubcoreMesh(core_axis_name="c", subcore_axis_name="s")

    @functools.partial(
        pl.kernel, mesh=mesh,
        out_type=jax.ShapeDtypeStruct((M, D), jnp.float32),
        scratch_types=[
            pltpu.VMEM((npw * k,), jnp.int32),
            pltpu.VMEM((chi, D), jnp.float32),
            pltpu.VMEM((ch, D), jnp.float32),
            pltpu.SemaphoreType.DMA,
        ],
    )
    def gmax(table_hbm, idx_hbm, out_hbm, idx_v, rows_v, out_v, sem):
        wid = lax.axis_index("s") * info.num_cores + lax.axis_index("c")
        pltpu.sync_copy(idx_hbm.at[pl.ds(wid * npw * k, npw * k)], idx_v)

        def body(c, carry):
            off = pl.multiple_of(c * chi, 8)
            pltpu.async_copy(table_hbm.at[idx_v.at[pl.ds(off, chi)]],
                             rows_v, sem).wait()
            for n in range(ch):
                for d in range(D // 16):
                    sl = pl.ds(d * 16, 16)
                    m = rows_v[n * k, sl]
                    for j in range(1, k):
                        m = jnp.maximum(m, rows_v[n * k + j, sl])
                    out_v[n, sl] = m
            row0 = pl.multiple_of(wid * npw + c * ch, 8)
            pltpu.sync_copy(out_v, out_hbm.at[pl.ds(row0, ch)])
            return carry

        lax.fori_loop(0, nch, body, 0)

    return gmax(table, idx)


def _out_kernel(ut_ref, m0_ref, x_ref, bg_ref, sg_ref, beg_ref, w2_ref, b2_ref,
                out_ref):
    hidden = ut_ref.shape[2]
    e = ut_ref[0] + m0_ref[0][:, :hidden] + bg_ref[...]
    m = e * sg_ref[...] + beg_ref[...]
    g = 0.5 * m * (1.0 + lax.erf(m * jnp.float32(1.0 / math.sqrt(2.0))))
    dn = (((1,), (1,)), ((), ()))
    out = lax.dot_general(w2_ref[...], g, dn, precision=_HIGH)
    out_ref[0] = out + b2_ref[...] + x_ref[0]


def kernel(x, W1, b1, g1, be1, Wg, bg, gg, beg, W2, b2, g2, be2):
    B, C, H, W = x.shape
    N = H * W
    hidden = Wg.shape[0]
    R = _ROW_BLOCK
    Np = -(-N // R) * R
    xf = x.reshape(B, C, N)
    if Np > N:
        xf = jnp.pad(xf, ((0, 0), (0, 0), (0, Np - N)))

    Hp = -(-hidden // 128) * 128
    inv = jnp.float32(1.0) / jnp.sqrt(jnp.float32(1.0 + _BN_EPS))
    b1c = b1[:, None]
    g1c = g1[:, None]
    be1c = be1[:, None]
    Wa = Wg[:, :C]
    Wb = Wg[:, C:]
    AT = jnp.transpose(Wa - Wb)
    WbT = jnp.pad(jnp.transpose(Wb), ((0, 0), (0, Hp - hidden)))
    sg = (gg * inv)[None, :]
    begr = beg[None, :]
    bgr = bg[None, :]
    s2 = g2 * inv
    W2f = W2 * s2[:, None]
    b2f = (b2 * s2 + be2)[:, None]

    full = lambda shape: pl.BlockSpec(shape, lambda b, *_: (0,) * len(shape))

    xn, uT, vT = pl.pallas_call(
        _fc1_kernel,
        grid=(B,),
        in_specs=[
            pl.BlockSpec((1, C, Np), lambda b: (b, 0, 0)),
            full((C, C)), full((C, 1)), full((C, 1)), full((C, 1)),
            full((C, hidden)), full((C, Hp)),
        ],
        out_specs=[
            pl.BlockSpec((1, C, Np), lambda b: (b, 0, 0)),
            pl.BlockSpec((1, Np, hidden), lambda b: (b, 0, 0)),
            pl.BlockSpec((1, Np, Hp), lambda b: (b, 0, 0)),
        ],
        out_shape=[
            jax.ShapeDtypeStruct((B, C, Np), jnp.float32),
            jax.ShapeDtypeStruct((B, Np, hidden), jnp.float32),
            jax.ShapeDtypeStruct((B, Np, Hp), jnp.float32),
        ],
    )(xf, W1, b1c, g1c, be1c, AT, WbT)

    nn_idx = pl.pallas_call(
        functools.partial(_knn_kernel, k=_K, n_real=N, n_tot=Np),
        grid=(B, Np // R),
        in_specs=[
            pl.BlockSpec((1, C, R), lambda b, r: (b, 0, r)),
            pl.BlockSpec((1, C, Np), lambda b, r: (b, 0, 0)),
        ],
        out_specs=pl.BlockSpec((1, R, _K), lambda b, r: (b, r, 0)),
        out_shape=jax.ShapeDtypeStruct((B, Np, _K), jnp.int32),
    )(xn, xn)

    m0T = _gather_max_sc(vT.reshape(B * Np, Hp),
                         nn_idx.reshape(B * Np * _K), _K)
    m0T = m0T.reshape(B, Np, Hp)

    out = pl.pallas_call(
        _out_kernel,
        grid=(B, Np // R),
        in_specs=[
            pl.BlockSpec((1, R, hidden), lambda b, r: (b, r, 0)),
            pl.BlockSpec((1, R, Hp), lambda b, r: (b, r, 0)),
            pl.BlockSpec((1, C, R), lambda b, r: (b, 0, r)),
            full((1, hidden)), full((1, hidden)), full((1, hidden)),
            full((C, hidden)), full((C, 1)),
        ],
        out_specs=pl.BlockSpec((1, C, R), lambda b, r: (b, 0, r)),
        out_shape=jax.ShapeDtypeStruct((B, C, Np), jnp.float32),
    )(uT, m0T, xf, bgr, sg, begr, W2f, b2f)

    return out[:, :, :N].reshape(B, C, H, W)


# speedup vs baseline: 1.2866x; 1.2866x over previous
import functools
import math

import jax
import jax.numpy as jnp
from jax import lax
from jax.experimental import pallas as pl
from jax.experimental.pallas import tpu as pltpu
from jax.experimental.pallas import tpu_sc as plsc

_BN_EPS = 1e-5
_K = 9
_HIGH = lax.Precision.HIGHEST


_ROW_BLOCK = 256


def _fc1_kernel(x_ref, w1_ref, b1_ref, g1_ref, be1_ref, at_ref, wbt_ref,
                xn_ref, ut_ref, vt_ref):
    xb = x_ref[0]
    h0 = lax.dot_general(w1_ref[...].astype(jnp.bfloat16),
                         xb.astype(jnp.bfloat16), (((1,), (0,)), ((), ())),
                         preferred_element_type=jnp.float32) + b1_ref[...]
    h = h0 / jnp.sqrt(jnp.float32(1.0 + _BN_EPS)) * g1_ref[...] + be1_ref[...]
    norm = jnp.sqrt(jnp.sum(h * h, axis=0, keepdims=True))
    xn_ref[0] = h / jnp.maximum(norm, 1e-12)
    dn = (((0,), (0,)), ((), ()))
    ut_ref[0] = lax.dot_general(h, at_ref[...], dn, precision=_HIGH)
    vt_ref[0] = lax.dot_general(h, wbt_ref[...], dn, precision=_HIGH)


def _knn_kernel(xnr_ref, xnf_ref, idx_ref, *, k, n_real, base):
    xr = xnr_ref[0]
    xf = xnf_ref[0]
    C, R = xr.shape
    N = xf.shape[1]
    dn = (((0,), (0,)), ((), ()))
    s = lax.dot_general(xr.astype(jnp.bfloat16), xf.astype(jnp.bfloat16), dn,
                        preferred_element_type=jnp.float32)
    xsq_col = jnp.sum(xf * xf, axis=0, keepdims=True)
    ones = jnp.ones((C, 1), jnp.float32)
    xsq_row = lax.dot_general(xr * xr, ones, dn, precision=_HIGH)
    dist = (xsq_row + (-2.0) * s) + xsq_col
    iota = lax.broadcasted_iota(jnp.int32, (R, N), 1)
    if n_real < N:
        dist = jnp.where(iota < n_real, dist, jnp.float32(jnp.inf))
    big = jnp.int32(2**30)
    inf = jnp.float32(jnp.inf)
    cols = []
    for _ in range(k):
        mn = jnp.min(dist, axis=1, keepdims=True)
        idx = jnp.min(jnp.where(dist <= mn, iota, big), axis=1, keepdims=True)
        cols.append(idx)
        dist = jnp.where(iota == idx, inf, dist)
    idx_ref[0] = jnp.concatenate(cols, axis=1) + jnp.int32(base)


def _gather_max_sc(table, idx, k):
    T, D = table.shape
    M = idx.shape[0] // k
    info = plsc.get_sparse_core_info()
    nw = info.num_cores * info.num_subcores
    npw = M // nw
    ch = 8
    nch = npw // ch
    chi = ch * k
    mesh = plsc.VectorSubcoreMesh(core_axis_name="c", subcore_axis_name="s")

    @functools.partial(
        pl.kernel, mesh=mesh,
        out_type=jax.ShapeDtypeStruct((M, D), jnp.float32),
        scratch_types=[
            pltpu.VMEM((npw * k,), jnp.int32),
            pltpu.VMEM((chi, D), jnp.float32),
            pltpu.VMEM((ch, D), jnp.float32),
            pltpu.SemaphoreType.DMA,
        ],
    )
    def gmax(table_hbm, idx_hbm, out_hbm, idx_v, rows_v, out_v, sem):
        wid = lax.axis_index("s") * info.num_cores + lax.axis_index("c")
        pltpu.sync_copy(idx_hbm.at[pl.ds(wid * npw * k, npw * k)], idx_v)

        def body(c, carry):
            off = pl.multiple_of(c * chi, 8)
            pltpu.async_copy(table_hbm.at[idx_v.at[pl.ds(off, chi)]],
                             rows_v, sem).wait()
            for n in range(ch):
                for d in range(D // 16):
                    sl = pl.ds(d * 16, 16)
                    m = rows_v[n * k, sl]
                    for j in range(1, k):
                        m = jnp.maximum(m, rows_v[n * k + j, sl])
                    out_v[n, sl] = m
            row0 = pl.multiple_of(wid * npw + c * ch, 8)
            pltpu.sync_copy(out_v, out_hbm.at[pl.ds(row0, ch)])
            return carry

        lax.fori_loop(0, nch, body, 0)

    return gmax(table, idx)


def _out_kernel(ut_ref, m0_ref, x_ref, bg_ref, sg_ref, beg_ref, w2_ref, b2_ref,
                out_ref):
    hidden = ut_ref.shape[2]
    e = ut_ref[0] + m0_ref[0][:, :hidden] + bg_ref[...]
    m = e * sg_ref[...] + beg_ref[...]
    g = 0.5 * m * (1.0 + lax.erf(m * jnp.float32(1.0 / math.sqrt(2.0))))
    dn = (((1,), (1,)), ((), ()))
    out = lax.dot_general(w2_ref[...], g, dn, precision=_HIGH)
    out_ref[0] = out + b2_ref[...] + x_ref[0]


def kernel(x, W1, b1, g1, be1, Wg, bg, gg, beg, W2, b2, g2, be2):
    B, C, H, W = x.shape
    N = H * W
    hidden = Wg.shape[0]
    R = _ROW_BLOCK
    Np = -(-N // R) * R
    xf = x.reshape(B, C, N)
    if Np > N:
        xf = jnp.pad(xf, ((0, 0), (0, 0), (0, Np - N)))

    Hp = -(-hidden // 128) * 128
    inv = jnp.float32(1.0) / jnp.sqrt(jnp.float32(1.0 + _BN_EPS))
    b1c = b1[:, None]
    g1c = g1[:, None]
    be1c = be1[:, None]
    Wa = Wg[:, :C]
    Wb = Wg[:, C:]
    AT = jnp.transpose(Wa - Wb)
    WbT = jnp.pad(jnp.transpose(Wb), ((0, 0), (0, Hp - hidden)))
    sg = (gg * inv)[None, :]
    begr = beg[None, :]
    bgr = bg[None, :]
    s2 = g2 * inv
    W2f = W2 * s2[:, None]
    b2f = (b2 * s2 + be2)[:, None]

    full = lambda shape: pl.BlockSpec(shape, lambda b, *_: (0,) * len(shape))

    xn, uT, vT = pl.pallas_call(
        _fc1_kernel,
        grid=(B,),
        in_specs=[
            pl.BlockSpec((1, C, Np), lambda b: (b, 0, 0)),
            full((C, C)), full((C, 1)), full((C, 1)), full((C, 1)),
            full((C, hidden)), full((C, Hp)),
        ],
        out_specs=[
            pl.BlockSpec((1, C, Np), lambda b: (b, 0, 0)),
            pl.BlockSpec((1, Np, hidden), lambda b: (b, 0, 0)),
            pl.BlockSpec((1, Np, Hp), lambda b: (b, 0, 0)),
        ],
        out_shape=[
            jax.ShapeDtypeStruct((B, C, Np), jnp.float32),
            jax.ShapeDtypeStruct((B, Np, hidden), jnp.float32),
            jax.ShapeDtypeStruct((B, Np, Hp), jnp.float32),
        ],
    )(xf, W1, b1c, g1c, be1c, AT, WbT)

    vflat = vT.reshape(B * Np, Hp)
    m0_parts = []
    for b in range(B):
        nn_idx_b = pl.pallas_call(
            functools.partial(_knn_kernel, k=_K, n_real=N, base=b * Np),
            grid=(Np // R,),
            in_specs=[
                pl.BlockSpec((1, C, R), lambda r, b=b: (b, 0, r)),
                pl.BlockSpec((1, C, Np), lambda r, b=b: (b, 0, 0)),
            ],
            out_specs=pl.BlockSpec((1, R, _K), lambda r: (0, r, 0)),
            out_shape=jax.ShapeDtypeStruct((1, Np, _K), jnp.int32),
        )(xn, xn)
        m0_parts.append(_gather_max_sc(vflat, nn_idx_b.reshape(Np * _K), _K))
    m0T = jnp.stack(m0_parts, axis=0)

    out = pl.pallas_call(
        _out_kernel,
        grid=(B, Np // R),
        in_specs=[
            pl.BlockSpec((1, R, hidden), lambda b, r: (b, r, 0)),
            pl.BlockSpec((1, R, Hp), lambda b, r: (b, r, 0)),
            pl.BlockSpec((1, C, R), lambda b, r: (b, 0, r)),
            full((1, hidden)), full((1, hidden)), full((1, hidden)),
            full((C, hidden)), full((C, 1)),
        ],
        out_specs=pl.BlockSpec((1, C, R), lambda b, r: (b, 0, r)),
        out_shape=jax.ShapeDtypeStruct((B, C, Np), jnp.float32),
    )(uT, m0T, xf, bgr, sg, begr, W2f, b2f)

    return out[:, :, :N].reshape(B, C, H, W)


# speedup vs baseline: 1.4559x; 1.1316x over previous
import functools
import math

import jax
import jax.numpy as jnp
from jax import lax
from jax.experimental import pallas as pl
from jax.experimental.pallas import tpu as pltpu
from jax.experimental.pallas import tpu_sc as plsc

_BN_EPS = 1e-5
_K = 9
_HIGH = lax.Precision.HIGHEST


_ROW_BLOCK = 256


def _fc1_kernel(x_ref, w1_ref, b1_ref, g1_ref, be1_ref, at_ref, wbt_ref,
                xn_ref, ut_ref, vt_ref):
    xb = x_ref[0]
    h0 = lax.dot_general(w1_ref[...].astype(jnp.bfloat16),
                         xb.astype(jnp.bfloat16), (((1,), (0,)), ((), ())),
                         preferred_element_type=jnp.float32) + b1_ref[...]
    h = h0 / jnp.sqrt(jnp.float32(1.0 + _BN_EPS)) * g1_ref[...] + be1_ref[...]
    norm = jnp.sqrt(jnp.sum(h * h, axis=0, keepdims=True))
    xn_ref[0] = h / jnp.maximum(norm, 1e-12)
    dn = (((0,), (0,)), ((), ()))
    ut_ref[0] = lax.dot_general(h, at_ref[...], dn)
    vt_ref[0] = lax.dot_general(h, wbt_ref[...], dn)


def _knn_kernel(xnr_ref, xnf_ref, idx_ref, *, k, n_real, base):
    xr = xnr_ref[0]
    xf = xnf_ref[0]
    C, R = xr.shape
    N = xf.shape[1]
    dn = (((0,), (0,)), ((), ()))
    s = lax.dot_general(xr.astype(jnp.bfloat16), xf.astype(jnp.bfloat16), dn,
                        preferred_element_type=jnp.float32)
    xsq_col = jnp.sum(xf * xf, axis=0, keepdims=True)
    ones = jnp.ones((C, 1), jnp.float32)
    xsq_row = lax.dot_general(xr * xr, ones, dn, precision=_HIGH)
    dist = (xsq_row + (-2.0) * s) + xsq_col
    iota = lax.broadcasted_iota(jnp.int32, (R, N), 1)
    if n_real < N:
        dist = jnp.where(iota < n_real, dist, jnp.float32(jnp.inf))
    big = jnp.int32(2**30)
    inf = jnp.float32(jnp.inf)
    cols = []
    for _ in range(k):
        mn = jnp.min(dist, axis=1, keepdims=True)
        sel = dist <= mn
        idx = jnp.min(jnp.where(sel, iota, big), axis=1, keepdims=True)
        cols.append(idx)
        dist = jnp.where(sel, inf, dist)
    idx_ref[0] = jnp.concatenate(cols, axis=1) + jnp.int32(base)


def _gather_max_sc(table, idx, k):
    T, D = table.shape
    M = idx.shape[0] // k
    info = plsc.get_sparse_core_info()
    nw = info.num_cores * info.num_subcores
    npw = M // nw
    ch = 8
    nch = npw // ch
    chi = ch * k
    mesh = plsc.VectorSubcoreMesh(core_axis_name="c", subcore_axis_name="s")

    @functools.partial(
        pl.kernel, mesh=mesh,
        out_type=jax.ShapeDtypeStruct((M, D), jnp.float32),
        scratch_types=[
            pltpu.VMEM((npw * k,), jnp.int32),
            pltpu.VMEM((chi, D), jnp.float32),
            pltpu.VMEM((ch, D), jnp.float32),
            pltpu.SemaphoreType.DMA,
        ],
    )
    def gmax(table_hbm, idx_hbm, out_hbm, idx_v, rows_v, out_v, sem):
        wid = lax.axis_index("s") * info.num_cores + lax.axis_index("c")
        pltpu.sync_copy(idx_hbm.at[pl.ds(wid * npw * k, npw * k)], idx_v)

        def body(c, carry):
            off = pl.multiple_of(c * chi, 8)
            pltpu.async_copy(table_hbm.at[idx_v.at[pl.ds(off, chi)]],
                             rows_v, sem).wait()
            for n in range(ch):
                for d in range(D // 16):
                    sl = pl.ds(d * 16, 16)
                    m = rows_v[n * k, sl]
                    for j in range(1, k):
                        m = jnp.maximum(m, rows_v[n * k + j, sl])
                    out_v[n, sl] = m
            row0 = pl.multiple_of(wid * npw + c * ch, 8)
            pltpu.sync_copy(out_v, out_hbm.at[pl.ds(row0, ch)])
            return carry

        lax.fori_loop(0, nch, body, 0)

    return gmax(table, idx)


def _out_kernel(ut_ref, m0_ref, x_ref, bg_ref, sg_ref, beg_ref, w2_ref, b2_ref,
                out_ref):
    hidden = ut_ref.shape[2]
    e = ut_ref[0] + m0_ref[0][:, :hidden] + bg_ref[...]
    m = e * sg_ref[...] + beg_ref[...]
    g = 0.5 * m * (1.0 + lax.erf(m * jnp.float32(1.0 / math.sqrt(2.0))))
    dn = (((1,), (1,)), ((), ()))
    out = lax.dot_general(w2_ref[...], g, dn)
    out_ref[0] = out + b2_ref[...] + x_ref[0]


def kernel(x, W1, b1, g1, be1, Wg, bg, gg, beg, W2, b2, g2, be2):
    B, C, H, W = x.shape
    N = H * W
    hidden = Wg.shape[0]
    R = _ROW_BLOCK
    Np = -(-N // R) * R
    xf = x.reshape(B, C, N)
    if Np > N:
        xf = jnp.pad(xf, ((0, 0), (0, 0), (0, Np - N)))

    Hp = -(-hidden // 128) * 128
    inv = jnp.float32(1.0) / jnp.sqrt(jnp.float32(1.0 + _BN_EPS))
    b1c = b1[:, None]
    g1c = g1[:, None]
    be1c = be1[:, None]
    Wa = Wg[:, :C]
    Wb = Wg[:, C:]
    AT = jnp.transpose(Wa - Wb)
    WbT = jnp.pad(jnp.transpose(Wb), ((0, 0), (0, Hp - hidden)))
    sg = (gg * inv)[None, :]
    begr = beg[None, :]
    bgr = bg[None, :]
    s2 = g2 * inv
    W2f = W2 * s2[:, None]
    b2f = (b2 * s2 + be2)[:, None]

    full = lambda shape: pl.BlockSpec(shape, lambda b, *_: (0,) * len(shape))

    xn, uT, vT = pl.pallas_call(
        _fc1_kernel,
        grid=(B,),
        in_specs=[
            pl.BlockSpec((1, C, Np), lambda b: (b, 0, 0)),
            full((C, C)), full((C, 1)), full((C, 1)), full((C, 1)),
            full((C, hidden)), full((C, Hp)),
        ],
        out_specs=[
            pl.BlockSpec((1, C, Np), lambda b: (b, 0, 0)),
            pl.BlockSpec((1, Np, hidden), lambda b: (b, 0, 0)),
            pl.BlockSpec((1, Np, Hp), lambda b: (b, 0, 0)),
        ],
        out_shape=[
            jax.ShapeDtypeStruct((B, C, Np), jnp.float32),
            jax.ShapeDtypeStruct((B, Np, hidden), jnp.float32),
            jax.ShapeDtypeStruct((B, Np, Hp), jnp.float32),
        ],
    )(xf, W1, b1c, g1c, be1c, AT, WbT)

    vflat = vT.reshape(B * Np, Hp)
    m0_parts = []
    for b in range(B):
        nn_idx_b = pl.pallas_call(
            functools.partial(_knn_kernel, k=_K, n_real=N, base=b * Np),
            grid=(Np // R,),
            in_specs=[
                pl.BlockSpec((1, C, R), lambda r, b=b: (b, 0, r)),
                pl.BlockSpec((1, C, Np), lambda r, b=b: (b, 0, 0)),
            ],
            out_specs=pl.BlockSpec((1, R, _K), lambda r: (0, r, 0)),
            out_shape=jax.ShapeDtypeStruct((1, Np, _K), jnp.int32),
        )(xn, xn)
        m0_parts.append(_gather_max_sc(vflat, nn_idx_b.reshape(Np * _K), _K))
    m0T = jnp.stack(m0_parts, axis=0)

    out = pl.pallas_call(
        _out_kernel,
        grid=(B, Np // R),
        in_specs=[
            pl.BlockSpec((1, R, hidden), lambda b, r: (b, r, 0)),
            pl.BlockSpec((1, R, Hp), lambda b, r: (b, r, 0)),
            pl.BlockSpec((1, C, R), lambda b, r: (b, 0, r)),
            full((1, hidden)), full((1, hidden)), full((1, hidden)),
            full((C, hidden)), full((C, 1)),
        ],
        out_specs=pl.BlockSpec((1, C, R), lambda b, r: (b, 0, r)),
        out_shape=jax.ShapeDtypeStruct((B, C, Np), jnp.float32),
    )(uT, m0T, xf, bgr, sg, begr, W2f, b2f)

    return out[:, :, :N].reshape(B, C, H, W)


# speedup vs baseline: 1.5734x; 1.0807x over previous
import functools
import math

import jax
import jax.numpy as jnp
from jax import lax
from jax.experimental import pallas as pl
from jax.experimental.pallas import tpu as pltpu
from jax.experimental.pallas import tpu_sc as plsc

_BN_EPS = 1e-5
_K = 9
_HIGH = lax.Precision.HIGHEST


_ROW_BLOCK = 256


def _fc1_kernel(x_ref, w1_ref, b1_ref, g1_ref, be1_ref, at_ref, wbt_ref,
                xn_ref, xsq_ref, ut_ref, vt_ref):
    xb = x_ref[0]
    h0 = lax.dot_general(w1_ref[...].astype(jnp.bfloat16),
                         xb.astype(jnp.bfloat16), (((1,), (0,)), ((), ())),
                         preferred_element_type=jnp.float32) + b1_ref[...]
    h = h0 / jnp.sqrt(jnp.float32(1.0 + _BN_EPS)) * g1_ref[...] + be1_ref[...]
    norm = jnp.sqrt(jnp.sum(h * h, axis=0, keepdims=True))
    xn = h / jnp.maximum(norm, 1e-12)
    xn_ref[0] = xn
    xsq_ref[0] = jnp.sum(xn * xn, axis=0, keepdims=True)
    dn = (((0,), (0,)), ((), ()))
    ut_ref[0] = lax.dot_general(h, at_ref[...], dn)
    vt_ref[0] = lax.dot_general(h, wbt_ref[...], dn)


def _knn_kernel(xnr_ref, xnf_ref, xsq_ref, idx_ref, *, k, n_real, base):
    xr = xnr_ref[0]
    xf = xnf_ref[0]
    C, R = xr.shape
    N = xf.shape[1]
    dn = (((0,), (0,)), ((), ()))
    s = lax.dot_general(xr.astype(jnp.bfloat16), xf.astype(jnp.bfloat16), dn,
                        preferred_element_type=jnp.float32)
    xsq_col = xsq_ref[0]
    ones = jnp.ones((C, 1), jnp.float32)
    xsq_row = lax.dot_general(xr * xr, ones, dn, precision=_HIGH)
    dist = (xsq_row + (-2.0) * s) + xsq_col
    iota = lax.broadcasted_iota(jnp.int32, (R, N), 1)
    if n_real < N:
        dist = jnp.where(iota < n_real, dist, jnp.float32(jnp.inf))
    big = jnp.int32(2**30)
    inf = jnp.float32(jnp.inf)
    cols = []
    for _ in range(k):
        idx = jnp.argmin(dist, axis=1).astype(jnp.int32)[:, None]
        cols.append(idx)
        dist = jnp.where(iota == idx, inf, dist)
    idx_ref[0] = jnp.concatenate(cols, axis=1) + jnp.int32(base)


def _gather_max_sc(table, idx, k):
    T, D = table.shape
    M = idx.shape[0] // k
    info = plsc.get_sparse_core_info()
    nw = info.num_cores * info.num_subcores
    npw = M // nw
    ch = 8
    nch = npw // ch
    chi = ch * k
    mesh = plsc.VectorSubcoreMesh(core_axis_name="c", subcore_axis_name="s")

    @functools.partial(
        pl.kernel, mesh=mesh,
        out_type=jax.ShapeDtypeStruct((M, D), jnp.float32),
        scratch_types=[
            pltpu.VMEM((npw * k,), jnp.int32),
            pltpu.VMEM((chi, D), jnp.float32),
            pltpu.VMEM((ch, D), jnp.float32),
            pltpu.SemaphoreType.DMA,
        ],
    )
    def gmax(table_hbm, idx_hbm, out_hbm, idx_v, rows_v, out_v, sem):
        wid = lax.axis_index("s") * info.num_cores + lax.axis_index("c")
        pltpu.sync_copy(idx_hbm.at[pl.ds(wid * npw * k, npw * k)], idx_v)

        def body(c, carry):
            off = pl.multiple_of(c * chi, 8)
            pltpu.async_copy(table_hbm.at[idx_v.at[pl.ds(off, chi)]],
                             rows_v, sem).wait()
            for n in range(ch):
                for d in range(D // 16):
                    sl = pl.ds(d * 16, 16)
                    m = rows_v[n * k, sl]
                    for j in range(1, k):
                        m = jnp.maximum(m, rows_v[n * k + j, sl])
                    out_v[n, sl] = m
            row0 = pl.multiple_of(wid * npw + c * ch, 8)
            pltpu.sync_copy(out_v, out_hbm.at[pl.ds(row0, ch)])
            return carry

        lax.fori_loop(0, nch, body, 0)

    return gmax(table, idx)


def _out_kernel(ut_ref, m0_ref, x_ref, bg_ref, sg_ref, beg_ref, w2_ref, b2_ref,
                out_ref):
    hidden = ut_ref.shape[2]
    e = ut_ref[0] + m0_ref[:, :hidden] + bg_ref[...]
    m = e * sg_ref[...] + beg_ref[...]
    g = 0.5 * m * (1.0 + lax.erf(m * jnp.float32(1.0 / math.sqrt(2.0))))
    dn = (((1,), (1,)), ((), ()))
    out = lax.dot_general(w2_ref[...], g, dn)
    out_ref[0] = out + b2_ref[...] + x_ref[0]


def kernel(x, W1, b1, g1, be1, Wg, bg, gg, beg, W2, b2, g2, be2):
    B, C, H, W = x.shape
    N = H * W
    hidden = Wg.shape[0]
    R = _ROW_BLOCK
    Np = -(-N // R) * R
    xf = x.reshape(B, C, N)
    if Np > N:
        xf = jnp.pad(xf, ((0, 0), (0, 0), (0, Np - N)))

    Hp = -(-hidden // 128) * 128
    inv = jnp.float32(1.0) / jnp.sqrt(jnp.float32(1.0 + _BN_EPS))
    b1c = b1[:, None]
    g1c = g1[:, None]
    be1c = be1[:, None]
    Wa = Wg[:, :C]
    Wb = Wg[:, C:]
    AT = jnp.transpose(Wa - Wb)
    WbT = jnp.pad(jnp.transpose(Wb), ((0, 0), (0, Hp - hidden)))
    sg = (gg * inv)[None, :]
    begr = beg[None, :]
    bgr = bg[None, :]
    s2 = g2 * inv
    W2f = W2 * s2[:, None]
    b2f = (b2 * s2 + be2)[:, None]

    full = lambda shape: pl.BlockSpec(shape, lambda b, *_: (0,) * len(shape))

    xn, xsq, uT, vT = pl.pallas_call(
        _fc1_kernel,
        grid=(B,),
        in_specs=[
            pl.BlockSpec((1, C, Np), lambda b: (b, 0, 0)),
            full((C, C)), full((C, 1)), full((C, 1)), full((C, 1)),
            full((C, hidden)), full((C, Hp)),
        ],
        out_specs=[
            pl.BlockSpec((1, C, Np), lambda b: (b, 0, 0)),
            pl.BlockSpec((1, 1, Np), lambda b: (b, 0, 0)),
            pl.BlockSpec((1, Np, hidden), lambda b: (b, 0, 0)),
            pl.BlockSpec((1, Np, Hp), lambda b: (b, 0, 0)),
        ],
        out_shape=[
            jax.ShapeDtypeStruct((B, C, Np), jnp.float32),
            jax.ShapeDtypeStruct((B, 1, Np), jnp.float32),
            jax.ShapeDtypeStruct((B, Np, hidden), jnp.float32),
            jax.ShapeDtypeStruct((B, Np, Hp), jnp.float32),
        ],
    )(xf, W1, b1c, g1c, be1c, AT, WbT)

    vflat = vT.reshape(B * Np, Hp)
    out_parts = []
    for b in range(B):
        nn_idx_b = pl.pallas_call(
            functools.partial(_knn_kernel, k=_K, n_real=N, base=b * Np),
            grid=(Np // R,),
            in_specs=[
                pl.BlockSpec((1, C, R), lambda r, b=b: (b, 0, r)),
                pl.BlockSpec((1, C, Np), lambda r, b=b: (b, 0, 0)),
                pl.BlockSpec((1, 1, Np), lambda r, b=b: (b, 0, 0)),
            ],
            out_specs=pl.BlockSpec((1, R, _K), lambda r: (0, r, 0)),
            out_shape=jax.ShapeDtypeStruct((1, Np, _K), jnp.int32),
        )(xn, xn, xsq)
        m0_b = _gather_max_sc(vflat, nn_idx_b.reshape(Np * _K), _K)
        out_parts.append(pl.pallas_call(
            _out_kernel,
            grid=(Np // R,),
            in_specs=[
                pl.BlockSpec((1, R, hidden), lambda r, b=b: (b, r, 0)),
                pl.BlockSpec((R, Hp), lambda r: (r, 0)),
                pl.BlockSpec((1, C, R), lambda r, b=b: (b, 0, r)),
                full((1, hidden)), full((1, hidden)), full((1, hidden)),
                full((C, hidden)), full((C, 1)),
            ],
            out_specs=pl.BlockSpec((1, C, R), lambda r: (0, 0, r)),
            out_shape=jax.ShapeDtypeStruct((1, C, Np), jnp.float32),
        )(uT, m0_b, xf, bgr, sg, begr, W2f, b2f))
    out = jnp.concatenate(out_parts, axis=0)

    return out[:, :, :N].reshape(B, C, H, W)


# speedup vs baseline: 1.6168x; 1.0276x over previous
import functools
import math

import jax
import jax.numpy as jnp
from jax import lax
from jax.experimental import pallas as pl
from jax.experimental.pallas import tpu as pltpu
from jax.experimental.pallas import tpu_sc as plsc

_BN_EPS = 1e-5
_K = 9
_HIGH = lax.Precision.HIGHEST


_ROW_BLOCK = 256


def _fc1_kernel(x_ref, w1_ref, b1_ref, g1_ref, be1_ref, at_ref, wbt_ref,
                xn_ref, xsq_ref, ut_ref, vt_ref):
    xb = x_ref[0]
    h0 = lax.dot_general(w1_ref[...].astype(jnp.bfloat16),
                         xb.astype(jnp.bfloat16), (((1,), (0,)), ((), ())),
                         preferred_element_type=jnp.float32) + b1_ref[...]
    h = h0 / jnp.sqrt(jnp.float32(1.0 + _BN_EPS)) * g1_ref[...] + be1_ref[...]
    norm = jnp.sqrt(jnp.sum(h * h, axis=0, keepdims=True))
    xn = h / jnp.maximum(norm, 1e-12)
    xn_ref[0] = xn
    xsq_ref[0] = jnp.sum(xn * xn, axis=0, keepdims=True)
    dn = (((0,), (0,)), ((), ()))
    ut_ref[0] = lax.dot_general(h, at_ref[...], dn)
    vt_ref[0] = lax.dot_general(h, wbt_ref[...], dn)


def _knn_kernel(xnr_ref, xnf_ref, xsq_ref, idx_ref, *, k, n_real, base):
    xr = xnr_ref[0]
    xf = xnf_ref[0]
    C, R = xr.shape
    N = xf.shape[1]
    dn = (((0,), (0,)), ((), ()))
    s = lax.dot_general(xr.astype(jnp.bfloat16), xf.astype(jnp.bfloat16), dn,
                        preferred_element_type=jnp.float32)
    xsq_col = xsq_ref[0]
    ones = jnp.ones((C, 1), jnp.float32)
    xsq_row = lax.dot_general(xr * xr, ones, dn, precision=_HIGH)
    dist = (xsq_row + (-2.0) * s) + xsq_col
    iota = lax.broadcasted_iota(jnp.int32, (R, N), 1)
    if n_real < N:
        dist = jnp.where(iota < n_real, dist, jnp.float32(jnp.inf))
    big = jnp.int32(2**30)
    inf = jnp.float32(jnp.inf)
    cols = []
    for _ in range(k):
        idx = jnp.argmin(dist, axis=1).astype(jnp.int32)[:, None]
        cols.append(idx)
        dist = jnp.where(iota == idx, inf, dist)
    idx_ref[0] = jnp.concatenate(cols, axis=1) + jnp.int32(base)


def _gather_max_sc(table, idx, k):
    T, D = table.shape
    M = idx.shape[0] // k
    info = plsc.get_sparse_core_info()
    nw = info.num_cores * info.num_subcores
    npw = M // nw
    ch = 8
    nch = npw // ch
    chi = ch * k
    mesh = plsc.VectorSubcoreMesh(core_axis_name="c", subcore_axis_name="s")

    assert nch % 2 == 1 and nch >= 3

    @functools.partial(
        pl.kernel, mesh=mesh,
        out_type=jax.ShapeDtypeStruct((M, D), jnp.float32),
        scratch_types=[
            pltpu.VMEM((npw * k,), jnp.int32),
            pltpu.VMEM((chi, D), jnp.float32),
            pltpu.VMEM((chi, D), jnp.float32),
            pltpu.VMEM((ch, D), jnp.float32),
            pltpu.SemaphoreType.DMA,
            pltpu.SemaphoreType.DMA,
        ],
    )
    def gmax(table_hbm, idx_hbm, out_hbm, idx_v, rows_a, rows_b, out_v,
             sem_a, sem_b):
        wid = lax.axis_index("s") * info.num_cores + lax.axis_index("c")
        pltpu.sync_copy(idx_hbm.at[pl.ds(wid * npw * k, npw * k)], idx_v)

        def fetch(c, buf, sem):
            off = pl.multiple_of(c * chi, 8)
            return pltpu.async_copy(table_hbm.at[idx_v.at[pl.ds(off, chi)]],
                                    buf, sem)

        def combine(c, rows_v):
            for n in range(ch):
                for d in range(D // 16):
                    sl = pl.ds(d * 16, 16)
                    m = rows_v[n * k, sl]
                    for j in range(1, k):
                        m = jnp.maximum(m, rows_v[n * k + j, sl])
                    out_v[n, sl] = m
            row0 = pl.multiple_of(wid * npw + c * ch, 8)
            pltpu.sync_copy(out_v, out_hbm.at[pl.ds(row0, ch)])

        fetch(0, rows_a, sem_a)

        def body2(c2, carry):
            c = c2 * 2
            fetch(c + 1, rows_b, sem_b)
            pltpu.make_async_copy(table_hbm.at[idx_v.at[pl.ds(0, chi)]],
                                  rows_a, sem_a).wait()
            combine(c, rows_a)
            fetch(c + 2, rows_a, sem_a)
            pltpu.make_async_copy(table_hbm.at[idx_v.at[pl.ds(0, chi)]],
                                  rows_b, sem_b).wait()
            combine(c + 1, rows_b)
            return carry

        lax.fori_loop(0, nch // 2, body2, 0)
        pltpu.make_async_copy(table_hbm.at[idx_v.at[pl.ds(0, chi)]],
                              rows_a, sem_a).wait()
        combine(nch - 1, rows_a)

    return gmax(table, idx)


def _out_kernel(ut_ref, m0_ref, x_ref, bg_ref, sg_ref, beg_ref, w2_ref, b2_ref,
                out_ref):
    hidden = ut_ref.shape[2]
    e = ut_ref[0] + m0_ref[:, :hidden] + bg_ref[...]
    m = e * sg_ref[...] + beg_ref[...]
    g = 0.5 * m * (1.0 + lax.erf(m * jnp.float32(1.0 / math.sqrt(2.0))))
    dn = (((1,), (1,)), ((), ()))
    out = lax.dot_general(w2_ref[...], g, dn)
    out_ref[0] = out + b2_ref[...] + x_ref[0]


def kernel(x, W1, b1, g1, be1, Wg, bg, gg, beg, W2, b2, g2, be2):
    B, C, H, W = x.shape
    N = H * W
    hidden = Wg.shape[0]
    R = _ROW_BLOCK
    Np = -(-N // R) * R
    xf = x.reshape(B, C, N)
    if Np > N:
        xf = jnp.pad(xf, ((0, 0), (0, 0), (0, Np - N)))

    Hp = -(-hidden // 128) * 128
    inv = jnp.float32(1.0) / jnp.sqrt(jnp.float32(1.0 + _BN_EPS))
    b1c = b1[:, None]
    g1c = g1[:, None]
    be1c = be1[:, None]
    Wa = Wg[:, :C]
    Wb = Wg[:, C:]
    AT = jnp.transpose(Wa - Wb)
    WbT = jnp.pad(jnp.transpose(Wb), ((0, 0), (0, Hp - hidden)))
    sg = (gg * inv)[None, :]
    begr = beg[None, :]
    bgr = bg[None, :]
    s2 = g2 * inv
    W2f = W2 * s2[:, None]
    b2f = (b2 * s2 + be2)[:, None]

    full = lambda shape: pl.BlockSpec(shape, lambda b, *_: (0,) * len(shape))

    xn, xsq, uT, vT = pl.pallas_call(
        _fc1_kernel,
        grid=(B,),
        in_specs=[
            pl.BlockSpec((1, C, Np), lambda b: (b, 0, 0)),
            full((C, C)), full((C, 1)), full((C, 1)), full((C, 1)),
            full((C, hidden)), full((C, Hp)),
        ],
        out_specs=[
            pl.BlockSpec((1, C, Np), lambda b: (b, 0, 0)),
            pl.BlockSpec((1, 1, Np), lambda b: (b, 0, 0)),
            pl.BlockSpec((1, Np, hidden), lambda b: (b, 0, 0)),
            pl.BlockSpec((1, Np, Hp), lambda b: (b, 0, 0)),
        ],
        out_shape=[
            jax.ShapeDtypeStruct((B, C, Np), jnp.float32),
            jax.ShapeDtypeStruct((B, 1, Np), jnp.float32),
            jax.ShapeDtypeStruct((B, Np, hidden), jnp.float32),
            jax.ShapeDtypeStruct((B, Np, Hp), jnp.float32),
        ],
    )(xf, W1, b1c, g1c, be1c, AT, WbT)

    vflat = vT.reshape(B * Np, Hp)
    out_parts = []
    for b in range(B):
        nn_idx_b = pl.pallas_call(
            functools.partial(_knn_kernel, k=_K, n_real=N, base=b * Np),
            grid=(Np // R,),
            in_specs=[
                pl.BlockSpec((1, C, R), lambda r, b=b: (b, 0, r)),
                pl.BlockSpec((1, C, Np), lambda r, b=b: (b, 0, 0)),
                pl.BlockSpec((1, 1, Np), lambda r, b=b: (b, 0, 0)),
            ],
            out_specs=pl.BlockSpec((1, R, _K), lambda r: (0, r, 0)),
            out_shape=jax.ShapeDtypeStruct((1, Np, _K), jnp.int32),
        )(xn, xn, xsq)
        m0_b = _gather_max_sc(vflat, nn_idx_b.reshape(Np * _K), _K)
        out_parts.append(pl.pallas_call(
            _out_kernel,
            grid=(Np // R,),
            in_specs=[
                pl.BlockSpec((1, R, hidden), lambda r, b=b: (b, r, 0)),
                pl.BlockSpec((R, Hp), lambda r: (r, 0)),
                pl.BlockSpec((1, C, R), lambda r, b=b: (b, 0, r)),
                full((1, hidden)), full((1, hidden)), full((1, hidden)),
                full((C, hidden)), full((C, 1)),
            ],
            out_specs=pl.BlockSpec((1, C, R), lambda r: (0, 0, r)),
            out_shape=jax.ShapeDtypeStruct((1, C, Np), jnp.float32),
        )(uT, m0_b, xf, bgr, sg, begr, W2f, b2f))
    out = jnp.concatenate(out_parts, axis=0)

    return out[:, :, :N].reshape(B, C, H, W)


# speedup vs baseline: 1.6230x; 1.0038x over previous
import functools
import math

import jax
import jax.numpy as jnp
from jax import lax
from jax.experimental import pallas as pl
from jax.experimental.pallas import tpu as pltpu
from jax.experimental.pallas import tpu_sc as plsc

_BN_EPS = 1e-5
_K = 9
_HIGH = lax.Precision.HIGHEST


_ROW_BLOCK = 256


def _fc1_kernel(x_ref, w1_ref, b1_ref, g1_ref, be1_ref, at_ref, wbt_ref,
                xn_ref, xsq_ref, ut_ref, vt_ref):
    xb = x_ref[0]
    h0 = lax.dot_general(w1_ref[...].astype(jnp.bfloat16),
                         xb.astype(jnp.bfloat16), (((1,), (0,)), ((), ())),
                         preferred_element_type=jnp.float32) + b1_ref[...]
    h = h0 / jnp.sqrt(jnp.float32(1.0 + _BN_EPS)) * g1_ref[...] + be1_ref[...]
    norm = jnp.sqrt(jnp.sum(h * h, axis=0, keepdims=True))
    xn = h / jnp.maximum(norm, 1e-12)
    xn_ref[0] = xn.astype(jnp.bfloat16)
    xsq_ref[0] = jnp.sum(xn * xn, axis=0, keepdims=True)
    dn = (((0,), (0,)), ((), ()))
    ut_ref[0] = lax.dot_general(h, at_ref[...], dn)
    vt_ref[0] = lax.dot_general(h, wbt_ref[...], dn)


def _knn_kernel(xnr_ref, xnf_ref, xsq_ref, idx_ref, *, k, n_real, base):
    xr = xnr_ref[0]
    xf = xnf_ref[0]
    C, R = xr.shape
    N = xf.shape[1]
    dn = (((0,), (0,)), ((), ()))
    s = lax.dot_general(xr, xf, dn,
                        preferred_element_type=jnp.float32)
    xsq_col = xsq_ref[0]
    xrf = xr.astype(jnp.float32)
    ones = jnp.ones((C, 1), jnp.float32)
    xsq_row = lax.dot_general(xrf * xrf, ones, dn, precision=_HIGH)
    dist = (xsq_row + (-2.0) * s) + xsq_col
    iota_row = lax.broadcasted_iota(jnp.int32, (1, N), 1)
    if n_real < N:
        dist = jnp.where(iota_row < n_real, dist, jnp.float32(jnp.inf))
    inf = jnp.float32(jnp.inf)
    cols = []
    for _ in range(k):
        idx = jnp.argmin(dist, axis=1).astype(jnp.int32)[:, None]
        cols.append(idx)
        dist = jnp.where(iota_row == idx, inf, dist)
    idx_ref[0] = jnp.concatenate(cols, axis=1) + jnp.int32(base)


def _gather_max_sc(table, idx, k):
    T, D = table.shape
    M = idx.shape[0] // k
    info = plsc.get_sparse_core_info()
    nw = info.num_cores * info.num_subcores
    npw = M // nw
    ch = 8
    nch = npw // ch
    chi = ch * k
    mesh = plsc.VectorSubcoreMesh(core_axis_name="c", subcore_axis_name="s")

    assert nch % 2 == 1 and nch >= 3

    @functools.partial(
        pl.kernel, mesh=mesh,
        out_type=jax.ShapeDtypeStruct((M, D), jnp.float32),
        scratch_types=[
            pltpu.VMEM((npw * k,), jnp.int32),
            pltpu.VMEM((chi, D), jnp.float32),
            pltpu.VMEM((chi, D), jnp.float32),
            pltpu.VMEM((ch, D), jnp.float32),
            pltpu.SemaphoreType.DMA,
            pltpu.SemaphoreType.DMA,
        ],
    )
    def gmax(table_hbm, idx_hbm, out_hbm, idx_v, rows_a, rows_b, out_v,
             sem_a, sem_b):
        wid = lax.axis_index("s") * info.num_cores + lax.axis_index("c")
        pltpu.sync_copy(idx_hbm.at[pl.ds(wid * npw * k, npw * k)], idx_v)

        def fetch(c, buf, sem):
            off = pl.multiple_of(c * chi, 8)
            return pltpu.async_copy(table_hbm.at[idx_v.at[pl.ds(off, chi)]],
                                    buf, sem)

        def combine(c, rows_v):
            for n in range(ch):
                for d in range(D // 16):
                    sl = pl.ds(d * 16, 16)
                    m = rows_v[n * k, sl]
                    for j in range(1, k):
                        m = jnp.maximum(m, rows_v[n * k + j, sl])
                    out_v[n, sl] = m
            row0 = pl.multiple_of(wid * npw + c * ch, 8)
            pltpu.sync_copy(out_v, out_hbm.at[pl.ds(row0, ch)])

        fetch(0, rows_a, sem_a)

        def body2(c2, carry):
            c = c2 * 2
            fetch(c + 1, rows_b, sem_b)
            pltpu.make_async_copy(table_hbm.at[idx_v.at[pl.ds(0, chi)]],
                                  rows_a, sem_a).wait()
            combine(c, rows_a)
            fetch(c + 2, rows_a, sem_a)
            pltpu.make_async_copy(table_hbm.at[idx_v.at[pl.ds(0, chi)]],
                                  rows_b, sem_b).wait()
            combine(c + 1, rows_b)
            return carry

        lax.fori_loop(0, nch // 2, body2, 0)
        pltpu.make_async_copy(table_hbm.at[idx_v.at[pl.ds(0, chi)]],
                              rows_a, sem_a).wait()
        combine(nch - 1, rows_a)

    return gmax(table, idx)


def _out_kernel(ut_ref, m0_ref, x_ref, bg_ref, sg_ref, beg_ref, w2_ref, b2_ref,
                out_ref):
    hidden = ut_ref.shape[2]
    e = ut_ref[0] + m0_ref[:, :hidden] + bg_ref[...]
    m = e * sg_ref[...] + beg_ref[...]
    g = 0.5 * m * (1.0 + lax.erf(m * jnp.float32(1.0 / math.sqrt(2.0))))
    dn = (((1,), (1,)), ((), ()))
    out = lax.dot_general(w2_ref[...], g, dn)
    out_ref[0] = out + b2_ref[...] + x_ref[0]


def kernel(x, W1, b1, g1, be1, Wg, bg, gg, beg, W2, b2, g2, be2):
    B, C, H, W = x.shape
    N = H * W
    hidden = Wg.shape[0]
    R = _ROW_BLOCK
    Np = -(-N // R) * R
    xf = x.reshape(B, C, N)
    if Np > N:
        xf = jnp.pad(xf, ((0, 0), (0, 0), (0, Np - N)))

    Hp = -(-hidden // 128) * 128
    inv = jnp.float32(1.0) / jnp.sqrt(jnp.float32(1.0 + _BN_EPS))
    b1c = b1[:, None]
    g1c = g1[:, None]
    be1c = be1[:, None]
    Wa = Wg[:, :C]
    Wb = Wg[:, C:]
    AT = jnp.transpose(Wa - Wb)
    WbT = jnp.pad(jnp.transpose(Wb), ((0, 0), (0, Hp - hidden)))
    sg = (gg * inv)[None, :]
    begr = beg[None, :]
    bgr = bg[None, :]
    s2 = g2 * inv
    W2f = W2 * s2[:, None]
    b2f = (b2 * s2 + be2)[:, None]

    full = lambda shape: pl.BlockSpec(shape, lambda b, *_: (0,) * len(shape))

    xn, xsq, uT, vT = pl.pallas_call(
        _fc1_kernel,
        grid=(B,),
        in_specs=[
            pl.BlockSpec((1, C, Np), lambda b: (b, 0, 0)),
            full((C, C)), full((C, 1)), full((C, 1)), full((C, 1)),
            full((C, hidden)), full((C, Hp)),
        ],
        out_specs=[
            pl.BlockSpec((1, C, Np), lambda b: (b, 0, 0)),
            pl.BlockSpec((1, 1, Np), lambda b: (b, 0, 0)),
            pl.BlockSpec((1, Np, hidden), lambda b: (b, 0, 0)),
            pl.BlockSpec((1, Np, Hp), lambda b: (b, 0, 0)),
        ],
        out_shape=[
            jax.ShapeDtypeStruct((B, C, Np), jnp.bfloat16),
            jax.ShapeDtypeStruct((B, 1, Np), jnp.float32),
            jax.ShapeDtypeStruct((B, Np, hidden), jnp.float32),
            jax.ShapeDtypeStruct((B, Np, Hp), jnp.float32),
        ],
    )(xf, W1, b1c, g1c, be1c, AT, WbT)

    vflat = vT.reshape(B * Np, Hp)
    out_parts = []
    for b in range(B):
        nn_idx_b = pl.pallas_call(
            functools.partial(_knn_kernel, k=_K, n_real=N, base=b * Np),
            grid=(Np // R,),
            in_specs=[
                pl.BlockSpec((1, C, R), lambda r, b=b: (b, 0, r)),
                pl.BlockSpec((1, C, Np), lambda r, b=b: (b, 0, 0)),
                pl.BlockSpec((1, 1, Np), lambda r, b=b: (b, 0, 0)),
            ],
            out_specs=pl.BlockSpec((1, R, _K), lambda r: (0, r, 0)),
            out_shape=jax.ShapeDtypeStruct((1, Np, _K), jnp.int32),
        )(xn, xn, xsq)
        m0_b = _gather_max_sc(vflat, nn_idx_b.reshape(Np * _K), _K)
        out_parts.append(pl.pallas_call(
            _out_kernel,
            grid=(Np // R,),
            in_specs=[
                pl.BlockSpec((1, R, hidden), lambda r, b=b: (b, r, 0)),
                pl.BlockSpec((R, Hp), lambda r: (r, 0)),
                pl.BlockSpec((1, C, R), lambda r, b=b: (b, 0, r)),
                full((1, hidden)), full((1, hidden)), full((1, hidden)),
                full((C, hidden)), full((C, 1)),
            ],
            out_specs=pl.BlockSpec((1, C, R), lambda r: (0, 0, r)),
            out_shape=jax.ShapeDtypeStruct((1, C, Np), jnp.float32),
        )(uT, m0_b, xf, bgr, sg, begr, W2f, b2f))
    out = jnp.concatenate(out_parts, axis=0)

    return out[:, :, :N].reshape(B, C, H, W)
